# baseline (speedup 1.0000x reference)
import functools

import jax
import jax.numpy as jnp
from jax import lax
from jax.experimental import pallas as pl
from jax.experimental.pallas import tpu as pltpu
from jax.experimental.pallas import tpu_sc as plsc

_NC = 2
_NS = 16
_NW = _NC * _NS
_LANES = 16
_CHUNK_ELEMS = 64 * 1024


def _sc_kernel(x_hbm, tbl_hbm, idx_hbm, out_hbm, idx_v, rows_v, buf,
               gsem, dsem, *, total, D):
    wid = lax.axis_index("s") * _NC + lax.axis_index("c")
    per_w = total // _NW
    base = wid * per_w

    pltpu.sync_copy(idx_hbm, idx_v)
    pltpu.async_copy(tbl_hbm.at[idx_v], rows_v, gsem).wait()

    n_chunks = per_w // _CHUNK_ELEMS

    def chunk_body(c, _):
        off = base + c * _CHUNK_ELEMS
        pltpu.async_copy(x_hbm.at[pl.ds(off, _CHUNK_ELEMS)], buf, dsem).wait()

        def vec_body(t, _):
            p = t * _LANES
            r = lax.rem(p, D)
            v = buf[pl.ds(p, _LANES)]
            buf[pl.ds(p, _LANES)] = v + rows_v[0, pl.ds(r, _LANES)]
            return _

        lax.fori_loop(0, _CHUNK_ELEMS // _LANES, vec_body, None)
        pltpu.async_copy(buf, out_hbm.at[pl.ds(off, _CHUNK_ELEMS)], dsem).wait()
        return _

    lax.fori_loop(0, n_chunks, chunk_body, None)


def kernel(feats, table, modality_id):
    B, S, D = feats.shape
    total = B * S * D
    x = feats.reshape(total)
    idx8 = jnp.full((8,), modality_id, dtype=jnp.int32)

    mesh = plsc.VectorSubcoreMesh(core_axis_name="c", subcore_axis_name="s")
    k = functools.partial(
        pl.kernel,
        mesh=mesh,
        out_type=jax.ShapeDtypeStruct((total,), jnp.float32),
        scratch_types=[
            pltpu.VMEM((8,), jnp.int32),
            pltpu.VMEM((8, D), jnp.float32),
            pltpu.VMEM((_CHUNK_ELEMS,), jnp.float32),
            pltpu.SemaphoreType.DMA,
            pltpu.SemaphoreType.DMA,
        ],
    )(functools.partial(_sc_kernel, total=total, D=D))
    out = k(x, table, idx8)
    return out.reshape(B, S, D)


# speedup vs baseline: 8.6917x; 8.6917x over previous
import jax
import jax.numpy as jnp
from jax.experimental import pallas as pl
from jax.experimental.pallas import tpu as pltpu


def _add_kernel(idx_ref, x1_ref, x2_ref, table_ref, out_ref):
    i = idx_ref[0]
    tbl = table_ref[...]
    rows = jax.lax.broadcasted_iota(jnp.int32, (tbl.shape[0], 1), 0)
    mask = (rows == i).astype(tbl.dtype)
    row = jnp.sum(tbl * mask, axis=0, keepdims=True)
    h = x1_ref.shape[0]
    out_ref[:h, :] = x1_ref[...] + row
    out_ref[h:, :] = x2_ref[...] + row


def kernel(feats, table, modality_id):
    B, S, D = feats.shape
    N = B * S
    x = feats.reshape(N, D)
    n_rows = table.shape[0]
    idx = jnp.asarray(modality_id, jnp.int32).reshape(1)

    rows_per_block = 2048
    half = rows_per_block // 2
    grid = (N // rows_per_block,)

    out = pl.pallas_call(
        _add_kernel,
        grid_spec=pltpu.PrefetchScalarGridSpec(
            num_scalar_prefetch=1,
            grid=grid,
            in_specs=[
                pl.BlockSpec((half, D), lambda i, idx_ref: (2 * i, 0)),
                pl.BlockSpec((half, D), lambda i, idx_ref: (2 * i + 1, 0)),
                pl.BlockSpec((n_rows, D), lambda i, idx_ref: (0, 0)),
            ],
            out_specs=pl.BlockSpec((rows_per_block, D), lambda i, idx_ref: (i, 0)),
        ),
        out_shape=jax.ShapeDtypeStruct((N, D), feats.dtype),
        compiler_params=pltpu.CompilerParams(
            dimension_semantics=("parallel",),
        ),
    )(idx, x, x, table)
    return out.reshape(B, S, D)
